# baseline (speedup 1.0000x reference)
import jax
import jax.numpy as jnp
import math
from jax.experimental import pallas as pl
from jax.experimental.pallas import tpu as pltpu

_B = 2
_N = 1000
_NP = 1024
_C = 91
_CP = 128
_IMG = 800.0
_SCORE_THRESH = 0.05
_NMS_THRESH = 0.5
_DETS = 100
_MIN_SIZE = 0.01
_CLIP = math.log(1000.0 / 16.0)
_BIG = 1 << 30


def _fused(lg_ref, d_ref, props_ref, out_ref, *scratch):
    f32 = jnp.float32
    refs = [scratch[b * 6:(b + 1) * 6] for b in range(_B)]
    l = lg_ref[...].reshape(_B, _CP, 8, 128)
    m = jnp.max(l, axis=1, keepdims=True)
    e = jnp.exp(l - m)
    p = e / jnp.sum(e, axis=1, keepdims=True)

    pr = props_ref[...].reshape(_B, 4, 1, 8, 128)
    p0 = pr[:, 0]
    p1 = pr[:, 1]
    p2 = pr[:, 2]
    p3 = pr[:, 3]
    w = (p2 - p0)
    h = (p3 - p1)
    cx = p0 + 0.5 * w
    cy = p1 + 0.5 * h
    d = d_ref[...].reshape(4, _B, _CP, 8, 128)
    dx = d[0] / 10.0
    dy = d[1] / 10.0
    dw = jnp.minimum(d[2] / 5.0, _CLIP)
    dh = jnp.minimum(d[3] / 5.0, _CLIP)
    pcx = dx * w + cx
    pcy = dy * h + cy
    pw = jnp.exp(dw) * w
    ph = jnp.exp(dh) * h
    x1 = jnp.clip(pcx - 0.5 * pw, 0.0, _IMG)
    y1 = jnp.clip(pcy - 0.5 * ph, 0.0, _IMG)
    x2 = jnp.clip(pcx + 0.5 * pw, 0.0, _IMG)
    y2 = jnp.clip(pcy + 0.5 * ph, 0.0, _IMG)

    crow = jax.lax.broadcasted_iota(jnp.int32, (_CP, 8, 128), 0)
    nidx = jax.lax.broadcasted_iota(jnp.int32, (_CP, 8, 128), 1) * 128 + \
        jax.lax.broadcasted_iota(jnp.int32, (_CP, 8, 128), 2)
    ok_pos = (crow >= 1) & (crow <= _C - 1) & (nidx < _N)
    offs = crow.astype(f32) * (_IMG + 2.0)

    lane = jax.lax.broadcasted_iota(jnp.int32, (1, _CP), 1)
    tidx = jax.lax.broadcasted_iota(jnp.int32, (8, 128), 0) * 128 + \
        jax.lax.broadcasted_iota(jnp.int32, (8, 128), 1)
    ones = jnp.ones((128, 128), f32)

    carries = []
    for b in range(_B):
        s_ref, xo1_ref, yo1_ref, xo2_ref, yo2_ref, ar_ref = refs[b]
        mask = (p[b] > _SCORE_THRESH) & ((x2[b] - x1[b]) >= _MIN_SIZE) & \
               ((y2[b] - y1[b]) >= _MIN_SIZE) & ok_pos
        s = jnp.where(mask, p[b], -1.0)
        s_ref[...] = s.reshape(_CP * 8, 128)
        xo1 = x1[b] + offs
        yo1 = y1[b] + offs
        xo2 = x2[b] + offs
        yo2 = y2[b] + offs
        xo1_ref[...] = xo1.reshape(_CP * 8, 128)
        yo1_ref[...] = yo1.reshape(_CP * 8, 128)
        xo2_ref[...] = xo2.reshape(_CP * 8, 128)
        yo2_ref[...] = yo2.reshape(_CP * 8, 128)
        ar_ref[...] = ((xo2 - xo1) * (yo2 - yo1)).reshape(_CP * 8, 128)
        M = jnp.max(s, axis=(1, 2)).reshape(1, _CP)
        mx0 = jnp.max(M)
        cc0 = jnp.min(jnp.where(M == mx0, lane, _BIG))
        carries.append((M, cc0, mx0))

    def one_image(b, carry, t):
        s_ref, xo1_ref, yo1_ref, xo2_ref, yo2_ref, ar_ref = refs[b]
        M, ccv, mxv = carry
        Mrest = jnp.where(lane == ccv, -2.0, M)
        mx_rest = jnp.max(Mrest)
        cc_rest = jnp.min(jnp.where(Mrest == mx_rest, lane, _BIG))

        base = ccv * 8
        srow = s_ref[pl.ds(base, 8), :]
        eq = srow == mxv
        ok = mxv > 0.0

        ro1 = xo1_ref[pl.ds(base, 8), :]
        ro2 = yo1_ref[pl.ds(base, 8), :]
        ro3 = xo2_ref[pl.ds(base, 8), :]
        ro4 = yo2_ref[pl.ds(base, 8), :]
        cat = jnp.concatenate(
            [jnp.where(eq, ro1, 0.0), jnp.where(eq, ro2, 0.0),
             jnp.where(eq, ro3, 0.0), jnp.where(eq, ro4, 0.0)], axis=0)
        sums = jax.lax.dot_general(cat, ones, (((1,), (0,)), ((), ())),
                                   precision=jax.lax.Precision.HIGHEST,
                                   preferred_element_type=f32)
        bs = jnp.sum(sums.reshape(4, 8, 128), axis=1)
        bx1 = jnp.broadcast_to(bs[0:1, :], (8, 128))
        by1 = jnp.broadcast_to(bs[1:2, :], (8, 128))
        bx2 = jnp.broadcast_to(bs[2:3, :], (8, 128))
        by2 = jnp.broadcast_to(bs[3:4, :], (8, 128))
        ref_area = (bx2 - bx1) * (by2 - by1)

        xx1 = jnp.maximum(bx1, ro1)
        yy1 = jnp.maximum(by1, ro2)
        xx2 = jnp.minimum(bx2, ro3)
        yy2 = jnp.minimum(by2, ro4)
        inter = jnp.maximum(xx2 - xx1, 0.0) * jnp.maximum(yy2 - yy1, 0.0)
        union = ar_ref[pl.ds(base, 8), :] + ref_area - inter
        iou = jnp.where(union > 0.0, inter / jnp.maximum(union, 1e-9), 0.0)
        supp = (iou > _NMS_THRESH) & ok
        srow_new = jnp.where(supp, -1.0, srow)
        s_ref[pl.ds(base, 8), :] = srow_new
        rowmax = jnp.max(srow_new)

        M_new = jnp.where(lane == ccv, rowmax, M)
        mx_next = jnp.maximum(mx_rest, rowmax)
        cc_next = jnp.where(rowmax >= mx_rest, ccv, cc_rest)

        nn = jnp.min(jnp.where(eq, tidx, _BIG))
        off_c = ccv.astype(f32) * (_IMG + 2.0)
        vf = jnp.where(ok, 1.0, 0.0).astype(f32)
        vals = [(bs[0, 0] - off_c) * vf, (bs[1, 0] - off_c) * vf,
                (bs[2, 0] - off_c) * vf, (bs[3, 0] - off_c) * vf,
                jnp.where(ok, mxv, 0.0),
                jnp.where(ok, ccv, 0).astype(f32),
                jnp.where(ok, nn * 90 + ccv - 1, 0).astype(f32),
                vf]
        for j, v in enumerate(vals):
            out_ref[b, pl.ds(t, 1), j:j + 1] = v.reshape(1, 1)
        return (M_new, cc_next, mx_next)

    def body(t, cs):
        return tuple(one_image(b, cs[b], t) for b in range(_B))

    jax.lax.fori_loop(0, _DETS, body, tuple(carries))


def kernel(class_logits, box_regression, proposals):
    f32 = jnp.float32
    lg = class_logits.astype(f32).reshape(_B, _N, _C).transpose(0, 2, 1)
    lg = jnp.pad(lg, ((0, 0), (0, _CP - _C), (0, _NP - _N)),
                 constant_values=-1e9).reshape(_B * _CP * 8, 128)
    d = box_regression.astype(f32).reshape(_B, _N, _C, 4).transpose(3, 0, 2, 1)
    d = jnp.pad(d, ((0, 0), (0, 0), (0, _CP - _C), (0, _NP - _N)))
    d = d.reshape(4 * _B * _CP * 8, 128)
    pr = proposals.astype(f32).transpose(0, 2, 1)
    pr = jnp.pad(pr, ((0, 0), (0, 0), (0, _NP - _N))).reshape(_B * 4 * 8, 128)

    out = pl.pallas_call(
        _fused,
        out_shape=jax.ShapeDtypeStruct((_B, _CP, _CP), f32),
        scratch_shapes=[pltpu.VMEM((_CP * 8, 128), f32)] * (6 * _B),
    )(lg, d, pr)

    res = out[:, :_DETS, :]
    sel_boxes = res[..., 0:4]
    sel_scores = res[..., 4]
    sel_labels = res[..., 5].astype(jnp.int32)
    keep = res[..., 6].astype(jnp.int32)
    valid = res[..., 7] > 0.5
    return sel_boxes, sel_scores, sel_labels, keep, valid


# speedup vs baseline: 1.0564x; 1.0564x over previous
import jax
import jax.numpy as jnp
import math
from jax.experimental import pallas as pl
from jax.experimental.pallas import tpu as pltpu

_B = 2
_N = 1000
_NP = 1024
_C = 91
_CP = 128
_IMG = 800.0
_SCORE_THRESH = 0.05
_NMS_THRESH = 0.5
_DETS = 100
_MIN_SIZE = 0.01
_CLIP = math.log(1000.0 / 16.0)
_BIG = 1 << 30


def _fused(lg_ref, d_ref, props_ref, out_ref, *scratch):
    f32 = jnp.float32
    refs = [scratch[b * 6:(b + 1) * 6] for b in range(_B)]
    lgn = lg_ref[...].reshape(_B, _NP, _CP)
    l = jnp.stack([jnp.transpose(lgn[b]).reshape(_CP, 8, 128)
                   for b in range(_B)])
    m = jnp.max(l, axis=1, keepdims=True)
    e = jnp.exp(l - m)
    p = e / jnp.sum(e, axis=1, keepdims=True)

    pr = props_ref[...].reshape(_B, 4, 1, 8, 128)
    p0 = pr[:, 0]
    p1 = pr[:, 1]
    p2 = pr[:, 2]
    p3 = pr[:, 3]
    w = (p2 - p0)
    h = (p3 - p1)
    cx = p0 + 0.5 * w
    cy = p1 + 0.5 * h
    dn = d_ref[...].reshape(_B, _NP, 512)
    dt = jnp.stack([jnp.transpose(dn[b]).reshape(_CP, 4, 8, 128)
                    for b in range(_B)])
    dx = dt[:, :, 0] / 10.0
    dy = dt[:, :, 1] / 10.0
    dw = jnp.minimum(dt[:, :, 2] / 5.0, _CLIP)
    dh = jnp.minimum(dt[:, :, 3] / 5.0, _CLIP)
    pcx = dx * w + cx
    pcy = dy * h + cy
    pw = jnp.exp(dw) * w
    ph = jnp.exp(dh) * h
    x1 = jnp.clip(pcx - 0.5 * pw, 0.0, _IMG)
    y1 = jnp.clip(pcy - 0.5 * ph, 0.0, _IMG)
    x2 = jnp.clip(pcx + 0.5 * pw, 0.0, _IMG)
    y2 = jnp.clip(pcy + 0.5 * ph, 0.0, _IMG)

    crow = jax.lax.broadcasted_iota(jnp.int32, (_CP, 8, 128), 0)
    nidx = jax.lax.broadcasted_iota(jnp.int32, (_CP, 8, 128), 1) * 128 + \
        jax.lax.broadcasted_iota(jnp.int32, (_CP, 8, 128), 2)
    ok_pos = (crow >= 1) & (crow <= _C - 1) & (nidx < _N)
    offs = crow.astype(f32) * (_IMG + 2.0)

    lane = jax.lax.broadcasted_iota(jnp.int32, (1, _CP), 1)
    tidx = jax.lax.broadcasted_iota(jnp.int32, (8, 128), 0) * 128 + \
        jax.lax.broadcasted_iota(jnp.int32, (8, 128), 1)
    ones = jnp.ones((128, 128), f32)

    carries = []
    for b in range(_B):
        s_ref, xo1_ref, yo1_ref, xo2_ref, yo2_ref, ar_ref = refs[b]
        mask = (p[b] > _SCORE_THRESH) & ((x2[b] - x1[b]) >= _MIN_SIZE) & \
               ((y2[b] - y1[b]) >= _MIN_SIZE) & ok_pos
        s = jnp.where(mask, p[b], -1.0)
        s_ref[...] = s.reshape(_CP * 8, 128)
        xo1 = x1[b] + offs
        yo1 = y1[b] + offs
        xo2 = x2[b] + offs
        yo2 = y2[b] + offs
        xo1_ref[...] = xo1.reshape(_CP * 8, 128)
        yo1_ref[...] = yo1.reshape(_CP * 8, 128)
        xo2_ref[...] = xo2.reshape(_CP * 8, 128)
        yo2_ref[...] = yo2.reshape(_CP * 8, 128)
        ar_ref[...] = ((xo2 - xo1) * (yo2 - yo1)).reshape(_CP * 8, 128)
        M = jnp.max(s, axis=(1, 2)).reshape(1, _CP)
        mx0 = jnp.max(M)
        cc0 = jnp.min(jnp.where(M == mx0, lane, _BIG))
        carries.append((M, cc0, mx0))

    def one_image(b, carry, t):
        s_ref, xo1_ref, yo1_ref, xo2_ref, yo2_ref, ar_ref = refs[b]
        M, ccv, mxv = carry
        Mrest = jnp.where(lane == ccv, -2.0, M)
        mx_rest = jnp.max(Mrest)
        cc_rest = jnp.min(jnp.where(Mrest == mx_rest, lane, _BIG))

        base = ccv * 8
        srow = s_ref[pl.ds(base, 8), :]
        eq = srow == mxv
        ok = mxv > 0.0

        ro1 = xo1_ref[pl.ds(base, 8), :]
        ro2 = yo1_ref[pl.ds(base, 8), :]
        ro3 = xo2_ref[pl.ds(base, 8), :]
        ro4 = yo2_ref[pl.ds(base, 8), :]
        cat = jnp.concatenate(
            [jnp.where(eq, ro1, 0.0), jnp.where(eq, ro2, 0.0),
             jnp.where(eq, ro3, 0.0), jnp.where(eq, ro4, 0.0)], axis=0)
        sums = jax.lax.dot_general(cat, ones, (((1,), (0,)), ((), ())),
                                   precision=jax.lax.Precision.HIGHEST,
                                   preferred_element_type=f32)
        bs = jnp.sum(sums.reshape(4, 8, 128), axis=1)
        bx1 = jnp.broadcast_to(bs[0:1, :], (8, 128))
        by1 = jnp.broadcast_to(bs[1:2, :], (8, 128))
        bx2 = jnp.broadcast_to(bs[2:3, :], (8, 128))
        by2 = jnp.broadcast_to(bs[3:4, :], (8, 128))
        ref_area = (bx2 - bx1) * (by2 - by1)

        xx1 = jnp.maximum(bx1, ro1)
        yy1 = jnp.maximum(by1, ro2)
        xx2 = jnp.minimum(bx2, ro3)
        yy2 = jnp.minimum(by2, ro4)
        inter = jnp.maximum(xx2 - xx1, 0.0) * jnp.maximum(yy2 - yy1, 0.0)
        union = ar_ref[pl.ds(base, 8), :] + ref_area - inter
        iou = jnp.where(union > 0.0, inter / jnp.maximum(union, 1e-9), 0.0)
        supp = (iou > _NMS_THRESH) & ok
        srow_new = jnp.where(supp, -1.0, srow)
        s_ref[pl.ds(base, 8), :] = srow_new
        rowmax = jnp.max(srow_new)

        M_new = jnp.where(lane == ccv, rowmax, M)
        mx_next = jnp.maximum(mx_rest, rowmax)
        cc_next = jnp.where(rowmax >= mx_rest, ccv, cc_rest)

        nn = jnp.min(jnp.where(eq, tidx, _BIG))
        off_c = ccv.astype(f32) * (_IMG + 2.0)
        vf = jnp.where(ok, 1.0, 0.0).astype(f32)
        vals = [(bs[0, 0] - off_c) * vf, (bs[1, 0] - off_c) * vf,
                (bs[2, 0] - off_c) * vf, (bs[3, 0] - off_c) * vf,
                jnp.where(ok, mxv, 0.0),
                jnp.where(ok, ccv, 0).astype(f32),
                jnp.where(ok, nn * 90 + ccv - 1, 0).astype(f32),
                vf]
        for j, v in enumerate(vals):
            out_ref[b, pl.ds(t, 1), j:j + 1] = v.reshape(1, 1)
        return (M_new, cc_next, mx_next)

    def body(t, cs):
        return tuple(one_image(b, cs[b], t) for b in range(_B))

    jax.lax.fori_loop(0, _DETS, body, tuple(carries))


def kernel(class_logits, box_regression, proposals):
    f32 = jnp.float32
    lg = jnp.pad(class_logits.astype(f32).reshape(_B, _N, _C),
                 ((0, 0), (0, _NP - _N), (0, _CP - _C)),
                 constant_values=-1e9).reshape(_B * _NP, _CP)
    d = jnp.pad(box_regression.astype(f32).reshape(_B, _N, _C * 4),
                ((0, 0), (0, _NP - _N), (0, 512 - _C * 4)))
    d = d.reshape(_B * _NP, 512)
    pr = proposals.astype(f32).transpose(0, 2, 1)
    pr = jnp.pad(pr, ((0, 0), (0, 0), (0, _NP - _N))).reshape(_B * 4 * 8, 128)

    out = pl.pallas_call(
        _fused,
        out_shape=jax.ShapeDtypeStruct((_B, _CP, _CP), f32),
        scratch_shapes=[pltpu.VMEM((_CP * 8, 128), f32)] * (6 * _B),
    )(lg, d, pr)

    res = out[:, :_DETS, :]
    sel_boxes = res[..., 0:4]
    sel_scores = res[..., 4]
    sel_labels = res[..., 5].astype(jnp.int32)
    keep = res[..., 6].astype(jnp.int32)
    valid = res[..., 7] > 0.5
    return sel_boxes, sel_scores, sel_labels, keep, valid
